# baseline (speedup 1.0000x reference)
import functools

import jax
import jax.numpy as jnp
from jax import lax
from jax.experimental import pallas as pl
from jax.experimental.pallas import tpu as pltpu
from jax.experimental.pallas import tpu_sc as plsc

DIM = 64
BATCH = 16384
NC = 2
NS = 16
NW = NC * NS
BPW = BATCH // NW
CH = 64
NCH = BPW // CH
L = 16


def _make_sc_kernel():
    mesh = plsc.VectorSubcoreMesh(core_axis_name="c", subcore_axis_name="s")

    scratch = (
        [pltpu.VMEM((NCH, CH), jnp.int32) for _ in range(3)]
        + [pltpu.VMEM((2, CH, DIM), jnp.float32) for _ in range(9)]
        + [pltpu.VMEM((BPW,), jnp.float32),
           pltpu.SemaphoreType.DMA,
           pltpu.SemaphoreType.DMA]
    )

    @functools.partial(
        pl.kernel,
        out_type=jax.ShapeDtypeStruct((BATCH,), jnp.float32),
        mesh=mesh,
        scratch_types=scratch,
        compiler_params=pltpu.CompilerParams(
            needs_layout_passes=False, use_tc_tiling_on_sc=False),
    )
    def tri_kernel(h_idx, r_idx, t_idx, e1, e2, e3, r1, r2, r3, out,
                   hi_v, ri_v, ti_v,
                   bh1, bh2, bh3, br1, br2, br3, bt1, bt2, bt3,
                   out_v, sem0, sem1):
        wid = lax.axis_index("s") * NC + lax.axis_index("c")

        base = wid * NCH
        pltpu.sync_copy(h_idx.at[pl.ds(base, NCH)], hi_v)
        pltpu.sync_copy(r_idx.at[pl.ds(base, NCH)], ri_v)
        pltpu.sync_copy(t_idx.at[pl.ds(base, NCH)], ti_v)

        sems = [sem0, sem1]
        gathers = [(e1, hi_v, bh1), (e2, hi_v, bh2), (e3, hi_v, bh3),
                   (r1, ri_v, br1), (r2, ri_v, br2), (r3, ri_v, br3),
                   (e1, ti_v, bt1), (e2, ti_v, bt2), (e3, ti_v, bt3)]

        def issue(g):
            slot = g % 2
            return [pltpu.async_copy(tbl.at[iv.at[g]], buf.at[slot], sems[slot])
                    for (tbl, iv, buf) in gathers]

        rows = [grp * L + lax.iota(jnp.int32, L) for grp in range(CH // L)]

        pending = {0: issue(0)}
        for g in range(NCH):
            if g + 1 < NCH:
                pending[g + 1] = issue(g + 1)
            for h in pending.pop(g):
                h.wait()
            slot = g % 2
            terms = [(bh1.at[slot], br1.at[slot], bt3.at[slot]),
                     (bh2.at[slot], br2.at[slot], bt2.at[slot]),
                     (bh3.at[slot], br3.at[slot], bt1.at[slot])]

            def body(d, accs):
                col = jnp.full((L,), 0, jnp.int32) + d
                new = []
                for grp in range(CH // L):
                    acc = accs[grp]
                    for (a, b, c) in terms:
                        acc = acc + (plsc.load_gather(a, [rows[grp], col])
                                     * plsc.load_gather(b, [rows[grp], col])
                                     * plsc.load_gather(c, [rows[grp], col]))
                    new.append(acc)
                return tuple(new)

            accs = lax.fori_loop(
                0, DIM, body,
                tuple(jnp.zeros((L,), jnp.float32) for _ in range(CH // L)))
            for grp in range(CH // L):
                out_v[pl.ds(g * CH + grp * L, L)] = accs[grp]

        pltpu.sync_copy(out_v, out.at[pl.ds(wid * BPW, BPW)])

    return tri_kernel


_tri_kernel = _make_sc_kernel()


@jax.jit
def kernel(triples, ent_v1, ent_v2, ent_v3, rel_v1, rel_v2, rel_v3):
    h_idx = triples[:, 0].reshape(NW * NCH, CH).astype(jnp.int32)
    r_idx = triples[:, 1].reshape(NW * NCH, CH).astype(jnp.int32)
    t_idx = triples[:, 2].reshape(NW * NCH, CH).astype(jnp.int32)
    return _tri_kernel(h_idx, r_idx, t_idx,
                       ent_v1, ent_v2, ent_v3, rel_v1, rel_v2, rel_v3)
